# baseline (speedup 1.0000x reference)
import jax
import jax.numpy as jnp
from jax import lax
from jax.experimental import pallas as pl
from jax.experimental.pallas import tpu as pltpu
from jax.experimental.pallas import tpu_sc as plsc

N = 10000
NPAD = 10240
E = 160000
CIN = 256
NC = 2
NQ = 4
CQ = CIN // (NC * NQ)
NS = 16
L = 16
G = 80
NJ = E // NS // G
NB = 5
BINS = NPAD // NS
SLAB = 80
BM = 1000

_GDN = lax.GatherDimensionNumbers(
    offset_dims=(), collapsed_slice_dims=(0,), start_index_map=(0,))


def _splat(v, r):
    idx = jnp.full((L, 1), r, dtype=jnp.int32)
    return lax.gather(v, idx, _GDN, (1,),
                      mode=lax.GatherScatterMode.PROMISE_IN_BOUNDS)


def _rsqrt(d):
    i = plsc.bitcast(d, jnp.int32)
    i = jnp.int32(0x5F3759DF) - (i >> 1)
    y = plsc.bitcast(i, jnp.float32)
    for _ in range(3):
        y = y * (1.5 - 0.5 * d * y * y)
    return y


def _sc_body(x_hbm, src_hbm, dst_hbm, y_hbm, hstage,
             src_my, dst_my, hist, hblk, dloc, dinv_v, rows, bufs,
             acc, ushared, dstage, gsem, ssem):
    cc = lax.axis_index("c")
    ss = lax.axis_index("s")
    zero16f = jnp.zeros((L,), jnp.float32)
    one16f = jnp.ones((L,), jnp.float32)

    pltpu.sync_copy(src_hbm.at[ss], src_my)
    pltpu.sync_copy(dst_hbm.at[ss], dst_my)

    def _zero(i, carry):
        hist[pl.ds(i * L, L)] = zero16f
        return carry
    lax.fori_loop(0, NPAD // L, _zero, 0)

    def _hist(j, carry):
        for k in range(G // L):
            d = dst_my[j, pl.ds(k * L, L)]
            plsc.addupdate_scatter(hist, [d], one16f)
        return carry
    lax.fori_loop(0, NJ, _hist, 0)

    pltpu.sync_copy(hist, hstage.at[cc, ss])
    plsc.subcore_barrier()
    for t in range(NS):
        pltpu.sync_copy(hstage.at[cc, t, pl.ds(ss * BINS, BINS)], hblk.at[t])

    def _dinv(g, carry):
        acc16 = one16f
        for t in range(NS):
            acc16 = acc16 + hblk[t, pl.ds(g * L, L)]
        dloc[pl.ds(g * L, L)] = _rsqrt(acc16)
        return carry
    lax.fori_loop(0, BINS // L, _dinv, 0)
    pltpu.sync_copy(dloc, dstage.at[pl.ds(ss * BINS, BINS)])
    plsc.subcore_barrier()
    pltpu.sync_copy(dstage, dinv_v)

    row0 = ss * BINS
    ng = jnp.minimum(N - row0, BINS) // SLAB

    def _scale_slab(r0, square):
        for h in range(SLAB // L):
            dv = dinv_v[pl.ds(r0 + h * L, L)]
            if square:
                dv = dv * dv
            for r in range(L):
                f = _splat(dv, r)
                for k in range(CQ // L):
                    sl = (h * L + r, pl.ds(k * L, L))
                    bufs[sl] = bufs[sl] * f

    def _hop():
        def _fire_g(j, b):
            pltpu.async_copy(ushared.at[src_my.at[j]], rows.at[b], gsem.at[b])

        def _wait_g(j, b):
            pltpu.make_async_copy(ushared.at[src_my.at[j]], rows.at[b],
                                  gsem.at[b]).wait()

        def _fire_s(j, b):
            pltpu.async_copy(rows.at[b], acc.at[dst_my.at[j]], ssem.at[b],
                             add=True)

        def _wait_s(j, b):
            pltpu.make_async_copy(rows.at[b], acc.at[dst_my.at[j]],
                                  ssem.at[b]).wait()

        for b in range(NB):
            _fire_g(b, b)

        def _ring(t, carry):
            j0 = t * NB
            for b in range(NB):
                _wait_g(j0 + b, b)
                _fire_s(j0 + b, b)
            for b in range(NB):
                _wait_s(j0 + b, b)
                _fire_g(j0 + NB + b, b)
            return carry
        lax.fori_loop(0, NJ // NB - 1, _ring, 0)

        j0 = NJ - NB
        for b in range(NB):
            _wait_g(j0 + b, b)
            _fire_s(j0 + b, b)
        for b in range(NB):
            _wait_s(j0 + b, b)

    def _pass(q, carry):
        col0 = cc * (NQ * CQ) + q * CQ

        def _p3(g, carry2):
            r0 = row0 + g * SLAB
            pltpu.sync_copy(x_hbm.at[pl.ds(r0, SLAB), pl.ds(col0, CQ)], bufs)
            _scale_slab(r0, False)
            pltpu.sync_copy(bufs, ushared.at[pl.ds(r0, SLAB)])
            pltpu.sync_copy(bufs, acc.at[pl.ds(r0, SLAB)])
            return carry2
        lax.fori_loop(0, ng, _p3, 0)
        plsc.subcore_barrier()

        _hop()
        plsc.subcore_barrier()

        def _p6(g, carry2):
            r0 = row0 + g * SLAB
            pltpu.sync_copy(acc.at[pl.ds(r0, SLAB)], bufs)
            _scale_slab(r0, True)
            pltpu.sync_copy(bufs, ushared.at[pl.ds(r0, SLAB)])
            pltpu.sync_copy(bufs, acc.at[pl.ds(r0, SLAB)])
            return carry2
        lax.fori_loop(0, ng, _p6, 0)
        plsc.subcore_barrier()

        _hop()
        plsc.subcore_barrier()

        def _p8(g, carry2):
            r0 = row0 + g * SLAB
            pltpu.sync_copy(acc.at[pl.ds(r0, SLAB)], bufs)
            _scale_slab(r0, False)
            pltpu.sync_copy(bufs, y_hbm.at[pl.ds(r0, SLAB), pl.ds(col0, CQ)])
            return carry2
        lax.fori_loop(0, ng, _p8, 0)
        plsc.subcore_barrier()
        return carry
    lax.fori_loop(0, NQ, _pass, 0)


def _mm_body(y_ref, w_ref, b_ref, o_ref):
    o_ref[...] = jnp.dot(y_ref[...], w_ref[...],
                         preferred_element_type=jnp.float32) + b_ref[...]


def kernel(x, edge_index, W, b):
    ei = edge_index.astype(jnp.int32)
    srcr = ei[0].reshape(NS, NJ, G)
    dstr = ei[1].reshape(NS, NJ, G)

    mesh = plsc.VectorSubcoreMesh(core_axis_name="c", subcore_axis_name="s")
    out_t = (jax.ShapeDtypeStruct((N, CIN), jnp.float32),
             jax.ShapeDtypeStruct((NC, NS, NPAD), jnp.float32))
    scratch = [
        pltpu.VMEM((NJ, G), jnp.int32),
        pltpu.VMEM((NJ, G), jnp.int32),
        pltpu.VMEM((NPAD,), jnp.float32),
        pltpu.VMEM((NS, BINS), jnp.float32),
        pltpu.VMEM((BINS,), jnp.float32),
        pltpu.VMEM((NPAD,), jnp.float32),
        pltpu.VMEM((NB, G, CQ), jnp.float32),
        pltpu.VMEM((SLAB, CQ), jnp.float32),
        pltpu.VMEM_SHARED((NPAD, CQ), jnp.float32),
        pltpu.VMEM_SHARED((NPAD, CQ), jnp.float32),
        pltpu.VMEM_SHARED((NPAD,), jnp.float32),
        pltpu.SemaphoreType.DMA((NB,)),
        pltpu.SemaphoreType.DMA((NB,)),
    ]
    sc = pl.kernel(_sc_body, out_type=out_t, mesh=mesh, scratch_types=scratch,
                   compiler_params=pltpu.CompilerParams(needs_layout_passes=False, use_tc_tiling_on_sc=False))
    y, _ = sc(x, srcr, dstr)

    out = pl.pallas_call(
        _mm_body,
        grid=(N // BM,),
        in_specs=[
            pl.BlockSpec((BM, CIN), lambda i: (i, 0)),
            pl.BlockSpec((CIN, CIN), lambda i: (0, 0)),
            pl.BlockSpec((1, CIN), lambda i: (0, 0)),
        ],
        out_specs=pl.BlockSpec((BM, CIN), lambda i: (i, 0)),
        out_shape=jax.ShapeDtypeStruct((N, CIN), jnp.float32),
    )(y, W, b.reshape(1, CIN))
    return out


# speedup vs baseline: 1.1192x; 1.1192x over previous
import jax
import jax.numpy as jnp
from jax import lax
from jax.experimental import pallas as pl
from jax.experimental.pallas import tpu as pltpu
from jax.experimental.pallas import tpu_sc as plsc

N = 10000
NPAD = 10240
E = 160000
EP = 163840
CIN = 256
NC = 2
NQ = 4
CQ = CIN // (NC * NQ)
NS = 16
L = 16
G = 128
NJ = EP // NS // G
NB = 8
BINS = NPAD // NS
SLAB = 80
BM = 1000

_GDN = lax.GatherDimensionNumbers(
    offset_dims=(), collapsed_slice_dims=(0,), start_index_map=(0,))


def _splat(v, r):
    idx = jnp.full((L, 1), r, dtype=jnp.int32)
    return lax.gather(v, idx, _GDN, (1,),
                      mode=lax.GatherScatterMode.PROMISE_IN_BOUNDS)


def _rsqrt(d):
    i = plsc.bitcast(d, jnp.int32)
    i = jnp.int32(0x5F3759DF) - (i >> 1)
    y = plsc.bitcast(i, jnp.float32)
    for _ in range(3):
        y = y * (1.5 - 0.5 * d * y * y)
    return y


def _sc_body(x_hbm, src_hbm, dst_hbm, y_hbm, hstage,
             src_my, dst_my, hist, hblk, dloc, dinv_v, rows, bufs,
             acc, ushared, gsem, ssem, rsem, wsem):
    cc = lax.axis_index("c")
    ss = lax.axis_index("s")
    zero16f = jnp.zeros((L,), jnp.float32)
    one16f = jnp.ones((L,), jnp.float32)

    pltpu.sync_copy(src_hbm.at[ss], src_my)
    pltpu.sync_copy(dst_hbm.at[ss], dst_my)

    def _zero(i, carry):
        hist[pl.ds(i * L, L)] = zero16f
        return carry
    lax.fori_loop(0, NPAD // L, _zero, 0)

    def _hist(j, carry):
        for k in range(G // L):
            d = dst_my[j, pl.ds(k * L, L)]
            plsc.addupdate_scatter(hist, [d], one16f)
        return carry
    lax.fori_loop(0, NJ, _hist, 0)

    pltpu.sync_copy(hist, hstage.at[cc, ss])
    plsc.subcore_barrier()
    for t in range(NS):
        pltpu.sync_copy(hstage.at[cc, t, pl.ds(ss * BINS, BINS)], hblk.at[t])

    def _dinv(g, carry):
        acc16 = one16f
        for t in range(NS):
            acc16 = acc16 + hblk[t, pl.ds(g * L, L)]
        dloc[pl.ds(g * L, L)] = _rsqrt(acc16)
        return carry
    lax.fori_loop(0, BINS // L, _dinv, 0)
    pltpu.sync_copy(dloc, hstage.at[cc, NS, pl.ds(ss * BINS, BINS)])
    plsc.subcore_barrier()
    pltpu.sync_copy(hstage.at[cc, NS], dinv_v)

    row0 = ss * BINS
    ng = jnp.minimum(N - row0, BINS) // SLAB

    def _scale_slab(b, r0, square):
        for h in range(SLAB // L):
            dv = dinv_v[pl.ds(r0 + h * L, L)]
            if square:
                dv = dv * dv
            for r in range(L):
                f = _splat(dv, r)
                for k in range(CQ // L):
                    sl = (b, h * L + r, pl.ds(k * L, L))
                    bufs[sl] = bufs[sl] * f

    def _phase(read_ref, write_refs, square):
        def _rd_fire(g, b):
            pltpu.async_copy(read_ref(row0 + g * SLAB), bufs.at[b],
                             rsem.at[b])

        def _rd_wait(g, b):
            pltpu.make_async_copy(read_ref(row0 + g * SLAB), bufs.at[b],
                                  rsem.at[b]).wait()

        def _wr_fire(g, b):
            for f in write_refs:
                pltpu.async_copy(bufs.at[b], f(row0 + g * SLAB), wsem.at[b])

        def _wr_wait(g, b):
            for f in write_refs:
                pltpu.make_async_copy(bufs.at[b], f(row0 + g * SLAB),
                                      wsem.at[b]).wait()

        _rd_fire(0, 0)

        def _body(g, carry):
            b = g % 2

            @pl.when(g >= 1)
            def _():
                _wr_wait(g - 1, 1 - b)

            @pl.when(g + 1 < ng)
            def _():
                _rd_fire(g + 1, 1 - b)

            _rd_wait(g, b)
            _scale_slab(b, row0 + g * SLAB, square)
            _wr_fire(g, b)
            return carry
        lax.fori_loop(0, ng, _body, 0)
        _wr_wait(ng - 1, (ng - 1) % 2)

    def _hop():
        def _fire_g(j, b):
            pltpu.async_copy(ushared.at[src_my.at[j]], rows.at[b], gsem.at[b])

        def _wait_g(j, b):
            pltpu.make_async_copy(ushared.at[src_my.at[j]], rows.at[b],
                                  gsem.at[b]).wait()

        def _fire_s(j, b):
            pltpu.async_copy(rows.at[b], acc.at[dst_my.at[j]], ssem.at[b],
                             add=True)

        def _wait_s(j, b):
            pltpu.make_async_copy(rows.at[b], acc.at[dst_my.at[j]],
                                  ssem.at[b]).wait()

        for b in range(NB):
            _fire_g(b, b)

        def _ring(t, carry):
            j0 = t * NB
            for b in range(NB):
                _wait_g(j0 + b, b)
                _fire_s(j0 + b, b)
            for b in range(NB):
                _wait_s(j0 + b, b)
                _fire_g(j0 + NB + b, b)
            return carry
        lax.fori_loop(0, NJ // NB - 1, _ring, 0)

        j0 = NJ - NB
        for b in range(NB):
            _wait_g(j0 + b, b)
            _fire_s(j0 + b, b)
        for b in range(NB):
            _wait_s(j0 + b, b)

    def _pass(q, carry):
        col0 = cc * (NQ * CQ) + q * CQ
        from_x = lambda r0: x_hbm.at[pl.ds(r0, SLAB), pl.ds(col0, CQ)]
        from_acc = lambda r0: acc.at[pl.ds(r0, SLAB)]
        to_ushared = lambda r0: ushared.at[pl.ds(r0, SLAB)]
        to_acc = from_acc
        to_y = lambda r0: y_hbm.at[pl.ds(r0, SLAB), pl.ds(col0, CQ)]

        _phase(from_x, [to_ushared, to_acc], False)
        plsc.subcore_barrier()

        _hop()
        plsc.subcore_barrier()

        _phase(from_acc, [to_ushared, to_acc], True)
        plsc.subcore_barrier()

        _hop()
        plsc.subcore_barrier()

        _phase(from_acc, [to_y], False)
        plsc.subcore_barrier()
        return carry
    lax.fori_loop(0, NQ, _pass, 0)


def _mm_body(y_ref, w_ref, b_ref, o_ref):
    o_ref[...] = jnp.dot(y_ref[...], w_ref[...],
                         preferred_element_type=jnp.float32) + b_ref[...]


def kernel(x, edge_index, W, b):
    ei = edge_index.astype(jnp.int32)
    pad = EP - E
    src_pad = jnp.zeros((pad,), jnp.int32)
    dst_pad = N + (jnp.arange(pad, dtype=jnp.int32) % (NPAD - N))
    srcr = jnp.concatenate([ei[0], src_pad]).reshape(NS, NJ, G)
    dstr = jnp.concatenate([ei[1], dst_pad]).reshape(NS, NJ, G)

    mesh = plsc.VectorSubcoreMesh(core_axis_name="c", subcore_axis_name="s")
    out_t = (jax.ShapeDtypeStruct((N, CIN), jnp.float32),
             jax.ShapeDtypeStruct((NC, NS + 1, NPAD), jnp.float32))
    scratch = [
        pltpu.VMEM((NJ, G), jnp.int32),
        pltpu.VMEM((NJ, G), jnp.int32),
        pltpu.VMEM((NPAD,), jnp.float32),
        pltpu.VMEM((NS, BINS), jnp.float32),
        pltpu.VMEM((BINS,), jnp.float32),
        pltpu.VMEM((NPAD,), jnp.float32),
        pltpu.VMEM((NB, G, CQ), jnp.float32),
        pltpu.VMEM((2, SLAB, CQ), jnp.float32),
        pltpu.VMEM_SHARED((NPAD, CQ), jnp.float32),
        pltpu.VMEM_SHARED((NPAD, CQ), jnp.float32),
        pltpu.SemaphoreType.DMA((NB,)),
        pltpu.SemaphoreType.DMA((NB,)),
        pltpu.SemaphoreType.DMA((2,)),
        pltpu.SemaphoreType.DMA((2,)),
    ]
    sc = pl.kernel(_sc_body, out_type=out_t, mesh=mesh, scratch_types=scratch,
                   compiler_params=pltpu.CompilerParams(needs_layout_passes=False, use_tc_tiling_on_sc=False))
    y, _ = sc(x, srcr, dstr)

    out = pl.pallas_call(
        _mm_body,
        grid=(N // BM,),
        in_specs=[
            pl.BlockSpec((BM, CIN), lambda i: (i, 0)),
            pl.BlockSpec((CIN, CIN), lambda i: (0, 0)),
            pl.BlockSpec((1, CIN), lambda i: (0, 0)),
        ],
        out_specs=pl.BlockSpec((BM, CIN), lambda i: (i, 0)),
        out_shape=jax.ShapeDtypeStruct((N, CIN), jnp.float32),
    )(y, W, b.reshape(1, CIN))
    return out
